# baseline (speedup 1.0000x reference)
import functools

import jax
import jax.numpy as jnp
from jax import lax
from jax.experimental import pallas as pl
from jax.experimental.pallas import tpu as pltpu
from jax.experimental.pallas import tpu_sc as plsc

_NC = 2
_NS = 16
_NW = _NC * _NS
_L = 16


@functools.lru_cache(maxsize=None)
def _build_sc_add(B, C, D, T, NB):
    b_per_w = B // _NW
    n_chunks = b_per_w // NB
    mesh = plsc.VectorSubcoreMesh(core_axis_name="core", subcore_axis_name="sub")

    @functools.partial(
        pl.kernel,
        mesh=mesh,
        out_type=jax.ShapeDtypeStruct((B, C, D), jnp.float32),
        scratch_types=[
            pltpu.VMEM((n_chunks, NB), jnp.int32),
            pltpu.VMEM((NB, C, D), jnp.float32),
            pltpu.VMEM((NB, D), jnp.float32),
            pltpu.SemaphoreType.DMA,
            pltpu.SemaphoreType.DMA,
        ],
    )
    def sc_add(x_hbm, t2_hbm, emb_hbm, out_hbm, idx_v, xbuf, ebuf, semx, seme):
        wid = lax.axis_index("sub") * _NC + lax.axis_index("core")
        row0 = wid * n_chunks
        pltpu.sync_copy(t2_hbm.at[pl.ds(row0, n_chunks)], idx_v)

        def chunk(j, carry):
            b0 = wid * b_per_w + j * NB
            cpx = pltpu.async_copy(x_hbm.at[pl.ds(b0, NB)], xbuf, semx)
            cpe = pltpu.async_copy(emb_hbm.at[idx_v.at[j]], ebuf, seme)
            cpx.wait()
            cpe.wait()

            def inner(k, c2):
                off = pl.ds(k * _L, _L)
                for bi in range(NB):
                    e = ebuf[bi, off]
                    for ci in range(C):
                        xbuf[bi, ci, off] = xbuf[bi, ci, off] + e
                return c2

            lax.fori_loop(0, D // _L, inner, 0, unroll=2)
            pltpu.sync_copy(xbuf, out_hbm.at[pl.ds(b0, NB)])
            return carry

        lax.fori_loop(0, n_chunks, chunk, 0)

    return sc_add


def kernel(x, t, embedding):
    B, C, d1, d2 = x.shape
    T = embedding.shape[0]
    D = d1 * d2
    NB = 2
    x2 = x.reshape(B, C, D)
    emb2 = embedding.reshape(T, D)
    t2 = t.reshape(B // NB, NB)
    out = _build_sc_add(B, C, D, T, NB)(x2, t2, emb2)
    return out.reshape(B, C, d1, d2)


# speedup vs baseline: 1.1296x; 1.1296x over previous
import functools

import jax
import jax.numpy as jnp
from jax import lax
from jax.experimental import pallas as pl
from jax.experimental.pallas import tpu as pltpu
from jax.experimental.pallas import tpu_sc as plsc

_NC = 2
_NS = 16
_NW = _NC * _NS
_L = 16
_NBUF = 2


@functools.lru_cache(maxsize=None)
def _build_sc_add(B, C, D, T):
    b_per_w = B // _NW
    n_chunks = b_per_w
    n_groups = n_chunks // _NBUF
    mesh = plsc.VectorSubcoreMesh(core_axis_name="core", subcore_axis_name="sub")

    @functools.partial(
        pl.kernel,
        mesh=mesh,
        out_type=jax.ShapeDtypeStruct((B, C, D), jnp.float32),
        scratch_types=(
            [pltpu.VMEM((n_chunks, 1), jnp.int32)]
            + [pltpu.VMEM((1, C, D), jnp.float32) for _ in range(_NBUF)]
            + [pltpu.VMEM((1, D), jnp.float32) for _ in range(_NBUF)]
            + [pltpu.VMEM((1, C, D), jnp.float32) for _ in range(_NBUF)]
            + [pltpu.SemaphoreType.DMA for _ in range(2 * _NBUF)]
        ),
    )
    def sc_add(x_hbm, t2_hbm, emb_hbm, out_hbm,
               idx_v, xb0, xb1, eb0, eb1, ob0, ob1,
               si0, si1, so0, so1):
        xb, eb, ob = (xb0, xb1), (eb0, eb1), (ob0, ob1)
        semi, semo = (si0, si1), (so0, so1)
        wid = lax.axis_index("sub") * _NC + lax.axis_index("core")
        base = wid * b_per_w
        pltpu.sync_copy(t2_hbm.at[pl.ds(base, n_chunks)], idx_v)

        def in_descs(j, s):
            row = pl.ds(base + j, 1)
            return (
                pltpu.make_async_copy(x_hbm.at[row], xb[s], semi[s]),
                pltpu.make_async_copy(emb_hbm.at[idx_v.at[j]], eb[s], semi[s]),
            )

        def out_desc(j, s):
            return pltpu.make_async_copy(ob[s], out_hbm.at[pl.ds(base + j, 1)],
                                         semo[s])

        for s in range(_NBUF):
            for d in in_descs(s, s):
                d.start()

        def group(g, carry):
            for s in range(_NBUF):
                j = g * _NBUF + s
                for d in in_descs(j, s):
                    d.wait()

                @pl.when(g > 0)
                def _wait_out():
                    out_desc(j - _NBUF, s).wait()

                def inner(k, c2):
                    off = pl.ds(k * _L, _L)
                    e = eb[s][0, off]
                    for ci in range(C):
                        ob[s][0, ci, off] = xb[s][0, ci, off] + e
                    return c2

                lax.fori_loop(0, D // _L, inner, 0, unroll=4)

                @pl.when(j + _NBUF < n_chunks)
                def _refill():
                    for d in in_descs(j + _NBUF, s):
                        d.start()

                out_desc(j, s).start()
            return carry

        lax.fori_loop(0, n_groups, group, 0)
        for s in range(_NBUF):
            out_desc(n_chunks - _NBUF + s, s).wait()

    return sc_add


def kernel(x, t, embedding):
    B, C, d1, d2 = x.shape
    T = embedding.shape[0]
    D = d1 * d2
    x2 = x.reshape(B, C, D)
    emb2 = embedding.reshape(T, D)
    t2 = t.reshape(B, 1)
    out = _build_sc_add(B, C, D, T)(x2, t2, emb2)
    return out.reshape(B, C, d1, d2)
